# Initial kernel scaffold; baseline (speedup 1.0000x reference)
import jax
import jax.numpy as jnp
from jax.experimental import pallas as pl


def kernel(x, edge_index, batch, W1, b1, g1, be1, W2, b2, g2, be2, W3, b3, g3, be3):
    raise NotImplementedError("write your pallas kernel here")


# speedup vs baseline: 8.0119x; 8.0119x over previous
import functools

import jax
import jax.numpy as jnp
from jax import lax
from jax.experimental import pallas as pl
from jax.experimental.pallas import tpu as pltpu
from jax.experimental.pallas import tpu_sc as plsc

N = 10000
E = 320000
D = 128
G = 64
EPS = 1e-5

NC = 2
NS = 16
NW = NC * NS
EPW = E // NW
B = 80
T = EPW // B
ZR = 632
ZT = N - 15 * ZR

R = 400
NBLK = N // R


def _zero_my_rows(zeros_hbm, acc, sid):
    start = pl.multiple_of(sid * ZR, 8)

    @pl.when(sid < NS - 1)
    def _():
        pltpu.sync_copy(zeros_hbm, acc.at[pl.ds(start, ZR)])

    @pl.when(sid == NS - 1)
    def _():
        pltpu.sync_copy(zeros_hbm.at[pl.ds(0, ZT)], acc.at[pl.ds(start, ZT)])


def _flush_my_rows(acc, out_hbm, cid, sid):
    start = pl.multiple_of(sid * ZR, 8)
    row0 = pl.multiple_of(cid * N + sid * ZR, 8)

    @pl.when(sid < NS - 1)
    def _():
        pltpu.sync_copy(acc.at[pl.ds(start, ZR)], out_hbm.at[pl.ds(row0, ZR)])

    @pl.when(sid == NS - 1)
    def _():
        pltpu.sync_copy(acc.at[pl.ds(start, ZT)], out_hbm.at[pl.ds(row0, ZT)])


def _sc_degree_kernel():
    mesh = plsc.VectorSubcoreMesh(core_axis_name="c", subcore_axis_name="s")

    @functools.partial(
        pl.kernel,
        out_type=jax.ShapeDtypeStruct((2 * N, D), jnp.float32),
        mesh=mesh,
        scratch_types=[
            pltpu.VMEM((B,), jnp.int32),
            pltpu.VMEM((B, D), jnp.float32),
            pltpu.VMEM_SHARED((N, D), jnp.float32),
        ],
    )
    def body(dst_hbm, ones_hbm, zeros_hbm, out_hbm, dst_v, ones_v, acc):
        cid = lax.axis_index("c")
        sid = lax.axis_index("s")
        wid = cid * NS + sid
        _zero_my_rows(zeros_hbm, acc, sid)
        pltpu.sync_copy(ones_hbm, ones_v)
        plsc.subcore_barrier()

        def step(t, carry):
            base = pl.multiple_of(wid * EPW + t * B, 8)
            pltpu.sync_copy(dst_hbm.at[pl.ds(base, B)], dst_v)
            pltpu.sync_copy(ones_v, acc.at[dst_v], add=True)
            return carry

        lax.fori_loop(0, T, step, 0)
        plsc.subcore_barrier()
        _flush_my_rows(acc, out_hbm, cid, sid)

    return body


def _sc_message_kernel():
    mesh = plsc.VectorSubcoreMesh(core_axis_name="c", subcore_axis_name="s")

    @functools.partial(
        pl.kernel,
        out_type=jax.ShapeDtypeStruct((2 * N, D), jnp.float32),
        mesh=mesh,
        scratch_types=[
            pltpu.VMEM((B,), jnp.int32),
            pltpu.VMEM((B,), jnp.int32),
            pltpu.VMEM((B, D), jnp.float32),
            pltpu.VMEM_SHARED((N, D), jnp.float32),
            pltpu.SemaphoreType.DMA,
        ],
    )
    def body(y_hbm, src_hbm, dst_hbm, zeros_hbm, out_hbm,
             src_v, dst_v, rows_v, acc, sem):
        cid = lax.axis_index("c")
        sid = lax.axis_index("s")
        wid = cid * NS + sid
        _zero_my_rows(zeros_hbm, acc, sid)
        plsc.subcore_barrier()

        def step(t, carry):
            base = pl.multiple_of(wid * EPW + t * B, 8)
            pltpu.sync_copy(src_hbm.at[pl.ds(base, B)], src_v)
            pltpu.sync_copy(dst_hbm.at[pl.ds(base, B)], dst_v)
            pltpu.async_copy(y_hbm.at[src_v], rows_v, sem).wait()
            pltpu.sync_copy(rows_v, acc.at[dst_v], add=True)
            return carry

        lax.fori_loop(0, T, step, 0)
        plsc.subcore_barrier()
        _flush_my_rows(acc, out_hbm, cid, sid)

    return body


def _sc_degree(dst, onesD, zerosD):
    return _sc_degree_kernel()(dst, onesD, zerosD)


def _sc_message(y, src, dst, zerosD):
    return _sc_message_kernel()(y, src, dst, zerosD)


_HI = lax.Precision.HIGHEST


def _mm1_body(x_ref, w_ref, d0_ref, d1_ref, y_ref, dinv_ref):
    deg = d0_ref[:, 0:1] + d1_ref[:, 0:1] + 1.0
    dinv = lax.rsqrt(deg)
    dinv_ref[...] = dinv
    y_ref[...] = lax.dot_general(
        x_ref[...], w_ref[...], (((1,), (0,)), ((), ())),
        precision=_HI, preferred_element_type=jnp.float32) * dinv


def _tc_mm1(x, W, degp):
    return pl.pallas_call(
        _mm1_body,
        grid=(NBLK,),
        in_specs=[
            pl.BlockSpec((R, D), lambda i: (i, 0)),
            pl.BlockSpec((D, D), lambda i: (0, 0)),
            pl.BlockSpec((R, D), lambda i: (i, 0)),
            pl.BlockSpec((R, D), lambda i: (i + NBLK, 0)),
        ],
        out_specs=[
            pl.BlockSpec((R, D), lambda i: (i, 0)),
            pl.BlockSpec((R, 1), lambda i: (i, 0)),
        ],
        out_shape=[
            jax.ShapeDtypeStruct((N, D), jnp.float32),
            jax.ShapeDtypeStruct((N, 1), jnp.float32),
        ],
    )(x, W, degp, degp)


def _combine_body(p0_ref, p1_ref, y_ref, dinv_ref, b_ref, pre_ref, st_ref):
    i = pl.program_id(0)
    pre = (p0_ref[...] + p1_ref[...] + y_ref[...]) * dinv_ref[...] + b_ref[...]
    pre_ref[...] = pre
    s1 = jnp.sum(pre, axis=0, keepdims=True)
    s2 = jnp.sum(pre * pre, axis=0, keepdims=True)
    rows = lax.broadcasted_iota(jnp.int32, (8, D), 0)
    contrib = jnp.where(rows == 0, s1, 0.0) + jnp.where(rows == 1, s2, 0.0)

    @pl.when(i == 0)
    def _():
        st_ref[...] = jnp.zeros((8, D), jnp.float32)

    st_ref[...] += contrib


def _tc_combine(msgp, y, dinv, b):
    return pl.pallas_call(
        _combine_body,
        grid=(NBLK,),
        in_specs=[
            pl.BlockSpec((R, D), lambda i: (i, 0)),
            pl.BlockSpec((R, D), lambda i: (i + NBLK, 0)),
            pl.BlockSpec((R, D), lambda i: (i, 0)),
            pl.BlockSpec((R, 1), lambda i: (i, 0)),
            pl.BlockSpec((1, D), lambda i: (0, 0)),
        ],
        out_specs=[
            pl.BlockSpec((R, D), lambda i: (i, 0)),
            pl.BlockSpec((8, D), lambda i: (0, 0)),
        ],
        out_shape=[
            jax.ShapeDtypeStruct((N, D), jnp.float32),
            jax.ShapeDtypeStruct((8, D), jnp.float32),
        ],
    )(msgp, msgp, y, dinv, b)


def _mm23_body(pre_ref, st_ref, g_ref, be_ref, dinv_ref, w_ref, y_ref):
    st = st_ref[...]
    mean = st[0:1, :] * (1.0 / N)
    var = st[1:2, :] * (1.0 / N) - mean * mean
    scale = g_ref[...] * lax.rsqrt(var + EPS)
    h = (pre_ref[...] - mean) * scale + be_ref[...]
    h = jnp.maximum(h, 0.0)
    y_ref[...] = lax.dot_general(
        h, w_ref[...], (((1,), (0,)), ((), ())),
        precision=_HI, preferred_element_type=jnp.float32) * dinv_ref[...]


def _tc_mm23(pre, st, g, be, dinv, W):
    return pl.pallas_call(
        _mm23_body,
        grid=(NBLK,),
        in_specs=[
            pl.BlockSpec((R, D), lambda i: (i, 0)),
            pl.BlockSpec((8, D), lambda i: (0, 0)),
            pl.BlockSpec((1, D), lambda i: (0, 0)),
            pl.BlockSpec((1, D), lambda i: (0, 0)),
            pl.BlockSpec((R, 1), lambda i: (i, 0)),
            pl.BlockSpec((D, D), lambda i: (0, 0)),
        ],
        out_specs=pl.BlockSpec((R, D), lambda i: (i, 0)),
        out_shape=jax.ShapeDtypeStruct((N, D), jnp.float32),
    )(pre, st, g, be, dinv, W)


def _pool_body(pre_ref, st_ref, g_ref, be_ref, batch_ref, out_ref,
               ssum_ref, scnt_ref, smax_ref):
    i = pl.program_id(0)
    st = st_ref[...]
    mean = st[0:1, :] * (1.0 / N)
    var = st[1:2, :] * (1.0 / N) - mean * mean
    scale = g_ref[...] * lax.rsqrt(var + EPS)
    hn = (pre_ref[...] - mean) * scale + be_ref[...]

    bcol = batch_ref[0]
    seg = lax.broadcasted_iota(jnp.int32, (R, G), 1)
    oh = (bcol == seg).astype(jnp.float32)

    @pl.when(i == 0)
    def _():
        ssum_ref[...] = jnp.zeros((G, D), jnp.float32)
        scnt_ref[...] = jnp.zeros((G, 1), jnp.float32)
        smax_ref[...] = jnp.full((G, D), -jnp.inf, jnp.float32)

    ssum_ref[...] += lax.dot_general(
        oh, hn, (((0,), (0,)), ((), ())),
        precision=_HI, preferred_element_type=jnp.float32)
    scnt_ref[...] += lax.dot_general(
        oh, jnp.ones((R, 1), jnp.float32), (((0,), (0,)), ((), ())),
        precision=_HI, preferred_element_type=jnp.float32)

    bmin = jnp.min(bcol)
    bmax = jnp.max(bcol)
    for g in range(G):
        @pl.when((bmin <= g) & (g <= bmax))
        def _():
            m = jnp.max(jnp.where(bcol == g, hn, -jnp.inf), axis=0,
                        keepdims=True)
            smax_ref[g:g + 1, :] = jnp.maximum(smax_ref[g:g + 1, :], m)

    @pl.when(i == NBLK - 1)
    def _():
        mx = smax_ref[...]
        mx = jnp.where(jnp.isfinite(mx), mx, 0.0)
        out_ref[...] = ssum_ref[...] / jnp.maximum(scnt_ref[...], 1.0) + mx


def _tc_pool(pre, st, g, be, batch3d):
    return pl.pallas_call(
        _pool_body,
        grid=(NBLK,),
        in_specs=[
            pl.BlockSpec((R, D), lambda i: (i, 0)),
            pl.BlockSpec((8, D), lambda i: (0, 0)),
            pl.BlockSpec((1, D), lambda i: (0, 0)),
            pl.BlockSpec((1, D), lambda i: (0, 0)),
            pl.BlockSpec((1, R, 1), lambda i: (i, 0, 0)),
        ],
        out_specs=pl.BlockSpec((G, D), lambda i: (0, 0)),
        out_shape=jax.ShapeDtypeStruct((G, D), jnp.float32),
        scratch_shapes=[
            pltpu.VMEM((G, D), jnp.float32),
            pltpu.VMEM((G, 1), jnp.float32),
            pltpu.VMEM((G, D), jnp.float32),
        ],
    )(pre, st, g, be, batch3d)


def kernel(x, edge_index, batch, W1, b1, g1, be1, W2, b2, g2, be2,
           W3, b3, g3, be3):
    src = edge_index[0]
    dst = edge_index[1]
    batch3d = batch.reshape(NBLK, R, 1)
    onesD = jnp.ones((B, D), jnp.float32)
    zerosD = jnp.zeros((ZR, D), jnp.float32)
    b1r, g1r, be1r = b1.reshape(1, D), g1.reshape(1, D), be1.reshape(1, D)
    b2r, g2r, be2r = b2.reshape(1, D), g2.reshape(1, D), be2.reshape(1, D)
    b3r, g3r, be3r = b3.reshape(1, D), g3.reshape(1, D), be3.reshape(1, D)

    degp = _sc_degree(dst, onesD, zerosD)
    y1, dinv = _tc_mm1(x, W1, degp)
    m1 = _sc_message(y1, src, dst, zerosD)
    pre1, st1 = _tc_combine(m1, y1, dinv, b1r)
    y2 = _tc_mm23(pre1, st1, g1r, be1r, dinv, W2)
    m2 = _sc_message(y2, src, dst, zerosD)
    pre2, st2 = _tc_combine(m2, y2, dinv, b2r)
    y3 = _tc_mm23(pre2, st2, g2r, be2r, dinv, W3)
    m3 = _sc_message(y3, src, dst, zerosD)
    pre3, st3 = _tc_combine(m3, y3, dinv, b3r)
    return _tc_pool(pre3, st3, g3r, be3r, batch3d)


# speedup vs baseline: 13.6666x; 1.7058x over previous
import functools

import jax
import jax.numpy as jnp
from jax import lax
from jax.experimental import pallas as pl
from jax.experimental.pallas import tpu as pltpu
from jax.experimental.pallas import tpu_sc as plsc

N = 10000
E = 320000
D = 128
G = 64
EPS = 1e-5

NC = 2
NS = 16
NW = NC * NS
EPW = E // NW
B = 80
T = EPW // B
ZR = 632
ZT = N - 15 * ZR

R = 400
NBLK = N // R


def _zero_my_rows(zeros_hbm, acc, sid):
    start = pl.multiple_of(sid * ZR, 8)

    @pl.when(sid < NS - 1)
    def _():
        pltpu.sync_copy(zeros_hbm, acc.at[pl.ds(start, ZR)])

    @pl.when(sid == NS - 1)
    def _():
        pltpu.sync_copy(zeros_hbm.at[pl.ds(0, ZT)], acc.at[pl.ds(start, ZT)])


def _flush_my_rows(acc, out_hbm, cid, sid):
    start = pl.multiple_of(sid * ZR, 8)
    row0 = pl.multiple_of(cid * N + sid * ZR, 8)

    @pl.when(sid < NS - 1)
    def _():
        pltpu.sync_copy(acc.at[pl.ds(start, ZR)], out_hbm.at[pl.ds(row0, ZR)])

    @pl.when(sid == NS - 1)
    def _():
        pltpu.sync_copy(acc.at[pl.ds(start, ZT)], out_hbm.at[pl.ds(row0, ZT)])


def _sc_degree_kernel():
    mesh = plsc.VectorSubcoreMesh(core_axis_name="c", subcore_axis_name="s")

    @functools.partial(
        pl.kernel,
        out_type=jax.ShapeDtypeStruct((2 * N, D), jnp.float32),
        mesh=mesh,
        scratch_types=[
            pltpu.VMEM((T, B), jnp.int32),
            pltpu.VMEM((B, D), jnp.float32),
            pltpu.VMEM_SHARED((N, D), jnp.float32),
            pltpu.SemaphoreType.DMA,
            pltpu.SemaphoreType.DMA,
        ],
    )
    def body(dst3_hbm, ones_hbm, zeros_hbm, out_hbm, dst_all, ones_v, acc,
             s0, s1):
        cid = lax.axis_index("c")
        sid = lax.axis_index("s")
        wid = cid * NS + sid
        _zero_my_rows(zeros_hbm, acc, sid)
        pltpu.sync_copy(dst3_hbm.at[wid], dst_all)
        pltpu.sync_copy(ones_hbm, ones_v)
        plsc.subcore_barrier()

        def s_issue(t, sem):
            pltpu.async_copy(ones_v, acc.at[dst_all.at[t]], sem, add=True)

        def s_wait(t, sem):
            pltpu.make_async_copy(ones_v, acc.at[dst_all.at[t]], sem).wait()

        s_issue(0, s0)
        s_issue(1, s1)

        def step(k, carry):
            t0, t1 = 2 * k, 2 * k + 1
            s_wait(t0, s0)

            @pl.when(t0 + 2 < T)
            def _():
                s_issue(t0 + 2, s0)

            s_wait(t1, s1)

            @pl.when(t1 + 2 < T)
            def _():
                s_issue(t1 + 2, s1)

            return carry

        lax.fori_loop(0, (T - 1) // 2, step, 0)
        s_wait(T - 1, s0)
        plsc.subcore_barrier()
        _flush_my_rows(acc, out_hbm, cid, sid)

    return body


def _sc_message_kernel():
    mesh = plsc.VectorSubcoreMesh(core_axis_name="c", subcore_axis_name="s")

    @functools.partial(
        pl.kernel,
        out_type=jax.ShapeDtypeStruct((2 * N, D), jnp.float32),
        mesh=mesh,
        scratch_types=[
            pltpu.VMEM((EPW,), jnp.int32),
            pltpu.VMEM((T, B), jnp.int32),
            pltpu.VMEM((B, D), jnp.float32),
            pltpu.VMEM((B, D), jnp.float32),
            pltpu.SemaphoreType.DMA,
            pltpu.SemaphoreType.DMA,
            pltpu.SemaphoreType.DMA,
            pltpu.SemaphoreType.DMA,
            pltpu.VMEM_SHARED((N, D), jnp.float32),
        ],
    )
    def body(y_hbm, src_hbm, dst3_hbm, zeros_hbm, out_hbm,
             src_all, dst_all, r0, r1, g0, g1, s0, s1, acc):
        cid = lax.axis_index("c")
        sid = lax.axis_index("s")
        wid = cid * NS + sid
        _zero_my_rows(zeros_hbm, acc, sid)
        pltpu.sync_copy(src_hbm.at[pl.ds(pl.multiple_of(wid * EPW, 8), EPW)],
                        src_all)
        pltpu.sync_copy(dst3_hbm.at[wid], dst_all)
        plsc.subcore_barrier()

        def src_slice(t):
            return src_all.at[pl.ds(pl.multiple_of(t * B, 8), B)]

        def g_issue(t, rbuf, sem):
            pltpu.async_copy(y_hbm.at[src_slice(t)], rbuf, sem)

        def g_wait(t, rbuf, sem):
            pltpu.make_async_copy(y_hbm.at[src_slice(t)], rbuf, sem).wait()

        def s_issue(t, rbuf, sem):
            pltpu.async_copy(rbuf, acc.at[dst_all.at[t]], sem, add=True)

        def s_wait(t, rbuf, sem):
            pltpu.make_async_copy(rbuf, acc.at[dst_all.at[t]], sem).wait()

        g_issue(0, r0, g0)
        g_issue(1, r1, g1)

        def step(k, carry):
            t0, t1 = 2 * k, 2 * k + 1
            g_wait(t0, r0, g0)
            s_issue(t0, r0, s0)
            g_wait(t1, r1, g1)
            s_issue(t1, r1, s1)
            s_wait(t0, r0, s0)

            @pl.when(t0 + 2 < T)
            def _():
                g_issue(t0 + 2, r0, g0)

            s_wait(t1, r1, s1)

            @pl.when(t1 + 2 < T)
            def _():
                g_issue(t1 + 2, r1, g1)

            return carry

        lax.fori_loop(0, (T - 1) // 2, step, 0)
        g_wait(T - 1, r0, g0)
        s_issue(T - 1, r0, s0)
        s_wait(T - 1, r0, s0)
        plsc.subcore_barrier()
        _flush_my_rows(acc, out_hbm, cid, sid)

    return body


def _sc_degree(dst2, ones2, zerosD):
    return _sc_degree_kernel()(dst2, ones2, zerosD)


def _sc_message(y, src, dst2, zerosD):
    return _sc_message_kernel()(y, src, dst2, zerosD)


_HI = lax.Precision.HIGHEST


def _mm1_body(x_ref, w_ref, d0_ref, d1_ref, y_ref, dinv_ref):
    deg = d0_ref[:, 0:1] + d1_ref[:, 0:1] + 1.0
    dinv = lax.rsqrt(deg)
    dinv_ref[...] = dinv
    y_ref[...] = lax.dot_general(
        x_ref[...], w_ref[...], (((1,), (0,)), ((), ())),
        precision=_HI, preferred_element_type=jnp.float32) * dinv


def _tc_mm1(x, W, degp):
    return pl.pallas_call(
        _mm1_body,
        grid=(NBLK,),
        in_specs=[
            pl.BlockSpec((R, D), lambda i: (i, 0)),
            pl.BlockSpec((D, D), lambda i: (0, 0)),
            pl.BlockSpec((R, D), lambda i: (i, 0)),
            pl.BlockSpec((R, D), lambda i: (i + NBLK, 0)),
        ],
        out_specs=[
            pl.BlockSpec((R, D), lambda i: (i, 0)),
            pl.BlockSpec((R, 1), lambda i: (i, 0)),
        ],
        out_shape=[
            jax.ShapeDtypeStruct((N, D), jnp.float32),
            jax.ShapeDtypeStruct((N, 1), jnp.float32),
        ],
    )(x, W, degp, degp)


def _combine_body(p0_ref, p1_ref, y_ref, dinv_ref, b_ref, pre_ref, st_ref):
    i = pl.program_id(0)
    pre = (p0_ref[...] + p1_ref[...] + y_ref[...]) * dinv_ref[...] + b_ref[...]
    pre_ref[...] = pre
    s1 = jnp.sum(pre, axis=0, keepdims=True)
    s2 = jnp.sum(pre * pre, axis=0, keepdims=True)
    rows = lax.broadcasted_iota(jnp.int32, (8, D), 0)
    contrib = jnp.where(rows == 0, s1, 0.0) + jnp.where(rows == 1, s2, 0.0)

    @pl.when(i == 0)
    def _():
        st_ref[...] = jnp.zeros((8, D), jnp.float32)

    st_ref[...] += contrib


def _tc_combine(msgp, y, dinv, b):
    return pl.pallas_call(
        _combine_body,
        grid=(NBLK,),
        in_specs=[
            pl.BlockSpec((R, D), lambda i: (i, 0)),
            pl.BlockSpec((R, D), lambda i: (i + NBLK, 0)),
            pl.BlockSpec((R, D), lambda i: (i, 0)),
            pl.BlockSpec((R, 1), lambda i: (i, 0)),
            pl.BlockSpec((1, D), lambda i: (0, 0)),
        ],
        out_specs=[
            pl.BlockSpec((R, D), lambda i: (i, 0)),
            pl.BlockSpec((8, D), lambda i: (0, 0)),
        ],
        out_shape=[
            jax.ShapeDtypeStruct((N, D), jnp.float32),
            jax.ShapeDtypeStruct((8, D), jnp.float32),
        ],
    )(msgp, msgp, y, dinv, b)


def _mm23_body(pre_ref, st_ref, g_ref, be_ref, dinv_ref, w_ref, y_ref):
    st = st_ref[...]
    mean = st[0:1, :] * (1.0 / N)
    var = st[1:2, :] * (1.0 / N) - mean * mean
    scale = g_ref[...] * lax.rsqrt(var + EPS)
    h = (pre_ref[...] - mean) * scale + be_ref[...]
    h = jnp.maximum(h, 0.0)
    y_ref[...] = lax.dot_general(
        h, w_ref[...], (((1,), (0,)), ((), ())),
        precision=_HI, preferred_element_type=jnp.float32) * dinv_ref[...]


def _tc_mm23(pre, st, g, be, dinv, W):
    return pl.pallas_call(
        _mm23_body,
        grid=(NBLK,),
        in_specs=[
            pl.BlockSpec((R, D), lambda i: (i, 0)),
            pl.BlockSpec((8, D), lambda i: (0, 0)),
            pl.BlockSpec((1, D), lambda i: (0, 0)),
            pl.BlockSpec((1, D), lambda i: (0, 0)),
            pl.BlockSpec((R, 1), lambda i: (i, 0)),
            pl.BlockSpec((D, D), lambda i: (0, 0)),
        ],
        out_specs=pl.BlockSpec((R, D), lambda i: (i, 0)),
        out_shape=jax.ShapeDtypeStruct((N, D), jnp.float32),
    )(pre, st, g, be, dinv, W)


def _pool_body(pre_ref, st_ref, g_ref, be_ref, batch_ref, out_ref,
               ssum_ref, scnt_ref, smax_ref):
    i = pl.program_id(0)
    st = st_ref[...]
    mean = st[0:1, :] * (1.0 / N)
    var = st[1:2, :] * (1.0 / N) - mean * mean
    scale = g_ref[...] * lax.rsqrt(var + EPS)
    hn = (pre_ref[...] - mean) * scale + be_ref[...]

    bcol = batch_ref[0]
    seg = lax.broadcasted_iota(jnp.int32, (R, G), 1)
    oh = (bcol == seg).astype(jnp.float32)

    @pl.when(i == 0)
    def _():
        ssum_ref[...] = jnp.zeros((G, D), jnp.float32)
        scnt_ref[...] = jnp.zeros((G, 1), jnp.float32)
        smax_ref[...] = jnp.full((G, D), -jnp.inf, jnp.float32)

    ssum_ref[...] += lax.dot_general(
        oh, hn, (((0,), (0,)), ((), ())),
        precision=_HI, preferred_element_type=jnp.float32)
    scnt_ref[...] += lax.dot_general(
        oh, jnp.ones((R, 1), jnp.float32), (((0,), (0,)), ((), ())),
        precision=_HI, preferred_element_type=jnp.float32)

    bmin = jnp.min(bcol)
    bmax = jnp.max(bcol)
    for g in range(G):
        @pl.when((bmin <= g) & (g <= bmax))
        def _():
            m = jnp.max(jnp.where(bcol == g, hn, -jnp.inf), axis=0,
                        keepdims=True)
            smax_ref[g:g + 1, :] = jnp.maximum(smax_ref[g:g + 1, :], m)

    @pl.when(i == NBLK - 1)
    def _():
        mx = smax_ref[...]
        mx = jnp.where(jnp.isfinite(mx), mx, 0.0)
        out_ref[...] = ssum_ref[...] / jnp.maximum(scnt_ref[...], 1.0) + mx


def _tc_pool(pre, st, g, be, batch3d):
    return pl.pallas_call(
        _pool_body,
        grid=(NBLK,),
        in_specs=[
            pl.BlockSpec((R, D), lambda i: (i, 0)),
            pl.BlockSpec((8, D), lambda i: (0, 0)),
            pl.BlockSpec((1, D), lambda i: (0, 0)),
            pl.BlockSpec((1, D), lambda i: (0, 0)),
            pl.BlockSpec((1, R, 1), lambda i: (i, 0, 0)),
        ],
        out_specs=pl.BlockSpec((G, D), lambda i: (0, 0)),
        out_shape=jax.ShapeDtypeStruct((G, D), jnp.float32),
        scratch_shapes=[
            pltpu.VMEM((G, D), jnp.float32),
            pltpu.VMEM((G, 1), jnp.float32),
            pltpu.VMEM((G, D), jnp.float32),
        ],
    )(pre, st, g, be, batch3d)


def kernel(x, edge_index, batch, W1, b1, g1, be1, W2, b2, g2, be2,
           W3, b3, g3, be3):
    src = edge_index[0]
    dst2 = edge_index[1].reshape(NW, T, B)
    batch3d = batch.reshape(NBLK, R, 1)
    ones2 = jnp.ones((B, D), jnp.float32)
    zerosD = jnp.zeros((ZR, D), jnp.float32)
    b1r, g1r, be1r = b1.reshape(1, D), g1.reshape(1, D), be1.reshape(1, D)
    b2r, g2r, be2r = b2.reshape(1, D), g2.reshape(1, D), be2.reshape(1, D)
    b3r, g3r, be3r = b3.reshape(1, D), g3.reshape(1, D), be3.reshape(1, D)

    degp = _sc_degree(dst2, ones2, zerosD)
    y1, dinv = _tc_mm1(x, W1, degp)
    m1 = _sc_message(y1, src, dst2, zerosD)
    pre1, st1 = _tc_combine(m1, y1, dinv, b1r)
    y2 = _tc_mm23(pre1, st1, g1r, be1r, dinv, W2)
    m2 = _sc_message(y2, src, dst2, zerosD)
    pre2, st2 = _tc_combine(m2, y2, dinv, b2r)
    y3 = _tc_mm23(pre2, st2, g2r, be2r, dinv, W3)
    m3 = _sc_message(y3, src, dst2, zerosD)
    pre3, st3 = _tc_combine(m3, y3, dinv, b3r)
    return _tc_pool(pre3, st3, g3r, be3r, batch3d)
